# baseline (speedup 1.0000x reference)
import jax
import jax.numpy as jnp
from jax import lax
from jax.experimental import pallas as pl
from jax.experimental.pallas import tpu as pltpu

_ORDER = 50
_EMBED = 64
_PAIR = 2 * _EMBED


def _body(idx_s, pos_s, table, out_v, rows_v, sem):
    copies = []
    for i in range(_ORDER):
        row = jnp.maximum(idx_s[i], 0)
        base = (row // 2) * _PAIR
        copies.append(pltpu.make_async_copy(
            table.at[pl.ds(base, _PAIR)], rows_v.at[i], sem))
    for cp in copies:
        cp.start()

    lane = lax.broadcasted_iota(jnp.int32, (1, _PAIR), 1)
    low_half = lane < _EMBED

    for cp in copies:
        cp.wait()

    acc = jnp.zeros((1, _PAIR), jnp.float32)
    denom = jnp.float32(0.0)
    for i in range(_ORDER):
        row = idx_s[i]
        wi = jnp.where(row != -1, pos_s[i], jnp.float32(0.0))
        denom = denom + wi
        odd = lax.rem(jnp.maximum(row, 0), 2) == 1
        wlo = jnp.where(odd, jnp.float32(0.0), wi)
        whi = wi - wlo
        wvec = jnp.where(low_half, wlo, whi)
        acc = acc + rows_v[pl.ds(i, 1), :] * wvec

    folded = acc[:, :_EMBED] + acc[:, _EMBED:]
    out_v[...] = folded / denom


@jax.jit
def kernel(partial_path_candidate, objects_embeds, positional_encoding):
    table_flat = objects_embeds.reshape(-1)
    out = pl.pallas_call(
        _body,
        out_shape=jax.ShapeDtypeStruct((1, _EMBED), jnp.float32),
        in_specs=[
            pl.BlockSpec(memory_space=pltpu.SMEM),
            pl.BlockSpec(memory_space=pltpu.SMEM),
            pl.BlockSpec(memory_space=pl.ANY),
        ],
        out_specs=pl.BlockSpec(memory_space=pltpu.VMEM),
        scratch_shapes=[
            pltpu.VMEM((_ORDER, _PAIR), jnp.float32),
            pltpu.SemaphoreType.DMA,
        ],
    )(partial_path_candidate, positional_encoding, table_flat)
    return out.reshape(_EMBED)


# speedup vs baseline: 2.0488x; 2.0488x over previous
import jax
import jax.numpy as jnp
from jax import lax
from jax.experimental import pallas as pl
from jax.experimental.pallas import tpu as pltpu

_ORDER = 50
_EMBED = 64
_SUB = 8


def _block_index_map(k):
    def index_map(i, idx_ref, pos_ref):
        return jnp.maximum(idx_ref[k], 0) // _SUB, 0
    return index_map


def _body(idx_ref, pos_ref, *blocks_and_out):
    blocks = blocks_and_out[:_ORDER]
    out_v = blocks_and_out[_ORDER]

    iota8 = lax.broadcasted_iota(jnp.int32, (_SUB, 1), 0)
    acc = jnp.zeros((_SUB, _EMBED), jnp.float32)
    denom = jnp.float32(0.0)
    for k in range(_ORDER):
        row = idx_ref[k]
        wi = jnp.where(row != -1, pos_ref[k], jnp.float32(0.0))
        denom = denom + wi
        rem = lax.rem(jnp.maximum(row, 0), _SUB)
        sel = jnp.where(iota8 == rem, wi, jnp.float32(0.0))
        acc = acc + blocks[k][...] * sel

    out_v[...] = jnp.sum(acc, axis=0, keepdims=True) / denom


@jax.jit
def kernel(partial_path_candidate, objects_embeds, positional_encoding):
    grid_spec = pltpu.PrefetchScalarGridSpec(
        num_scalar_prefetch=2,
        grid=(1,),
        in_specs=[
            pl.BlockSpec((_SUB, _EMBED), _block_index_map(k))
            for k in range(_ORDER)
        ],
        out_specs=pl.BlockSpec((1, _EMBED), lambda i, idx_ref, pos_ref: (0, 0)),
    )
    out = pl.pallas_call(
        _body,
        grid_spec=grid_spec,
        out_shape=jax.ShapeDtypeStruct((1, _EMBED), jnp.float32),
    )(partial_path_candidate, positional_encoding,
      *([objects_embeds] * _ORDER))
    return out.reshape(_EMBED)


# speedup vs baseline: 17.4529x; 8.5185x over previous
import jax
import jax.numpy as jnp
from jax import lax
from jax.experimental import pallas as pl
from jax.experimental.pallas import tpu as pltpu

_ORDER = 50
_EMBED = 64
_LANES = 128


def _block_index_map(k):
    def index_map(i, idx_ref, pos_ref):
        return 0, jnp.maximum(idx_ref[k], 0) // _LANES
    return index_map


def _body(idx_ref, pos_ref, *blocks_and_out):
    blocks = blocks_and_out[:_ORDER]
    out_v = blocks_and_out[_ORDER]

    lane = lax.broadcasted_iota(jnp.int32, (1, _LANES), 1)
    acc = jnp.zeros((_EMBED, _LANES), jnp.float32)
    denom = jnp.float32(0.0)
    for k in range(_ORDER):
        row = idx_ref[k]
        wi = jnp.where(row != -1, pos_ref[k], jnp.float32(0.0))
        denom = denom + wi
        rem = lax.rem(jnp.maximum(row, 0), _LANES)
        sel = jnp.where(lane == rem, wi, jnp.float32(0.0))
        acc = acc + blocks[k][...] * sel

    out_v[...] = jnp.sum(acc, axis=1, keepdims=True) / denom


@jax.jit
def kernel(partial_path_candidate, objects_embeds, positional_encoding):
    table_t = objects_embeds.T
    grid_spec = pltpu.PrefetchScalarGridSpec(
        num_scalar_prefetch=2,
        grid=(1,),
        in_specs=[
            pl.BlockSpec((_EMBED, _LANES), _block_index_map(k))
            for k in range(_ORDER)
        ],
        out_specs=pl.BlockSpec((_EMBED, 1), lambda i, idx_ref, pos_ref: (0, 0)),
    )
    out = pl.pallas_call(
        _body,
        grid_spec=grid_spec,
        out_shape=jax.ShapeDtypeStruct((_EMBED, 1), jnp.float32),
    )(partial_path_candidate, positional_encoding, *([table_t] * _ORDER))
    return out.reshape(_EMBED)
